# baseline (speedup 1.0000x reference)
import functools

import jax
import jax.numpy as jnp
from jax import lax
from jax.experimental import pallas as pl
from jax.experimental.pallas import tpu as pltpu

_PREC = lax.Precision.HIGHEST
_KNN = 20
_E1 = float(1000.0 / 3.0)
_E2 = float(2000.0 / 3.0)


def _argmax_minor(x):
    n, c = x.shape
    mx = jnp.max(x, axis=1, keepdims=True)
    idx = lax.broadcasted_iota(jnp.int32, (n, c), 1).astype(jnp.float32)
    cand = jnp.where(x == mx, idx, float(c))
    return jnp.min(cand, axis=1, keepdims=True).astype(jnp.int32)


def _prologue_body(coords_ref, feat_ref, cls_ref, protof_ref, p1_ref, p2_ref):
    f = feat_ref[...]
    logits1 = lax.dot_general(f, cls_ref[...], (((1,), (0,)), ((), ())),
                              precision=_PREC)
    p1_ref[...] = _argmax_minor(logits1)

    c = coords_ref[...]
    d = jnp.sqrt(c[:, 0:1] * c[:, 0:1] + c[:, 1:2] * c[:, 1:2])
    d = jnp.clip(d, 0.001, 999.999)
    lab = (d >= _E1).astype(jnp.int32) + (d >= _E2).astype(jnp.int32)

    la = lax.dot_general(f, protof_ref[...], (((1,), (1,)), ((), ())),
                         precision=_PREC)
    nc = la.shape[1] // 3
    pp = jnp.where(lab == 0, la[:, 0:nc],
                   jnp.where(lab == 1, la[:, nc:2 * nc], la[:, 2 * nc:3 * nc]))
    p2_ref[...] = _argmax_minor(pp)


def _knn_body(coords_r_ref, coords_t_ref, p1r_ref, p2r_ref, p1c_ref, p2c_ref,
              mask_ref, d2_ref, *, n_cols, cw):
    cr = coords_r_ref[...]
    ct = coords_t_ref[...]
    sq_r = jnp.sum(cr * cr, axis=1, keepdims=True)
    sq_c = jnp.sum(ct * ct, axis=0, keepdims=True)
    dot = lax.dot_general(cr, ct, (((1,), (0,)), ((), ())), precision=_PREC)
    d2_ref[...] = sq_r + sq_c - 2.0 * dot

    rows = cr.shape[0]
    nch = n_cols // cw
    nb = 16

    lo = jnp.full((rows, 1), -1.0, jnp.float32)
    cnt = jnp.zeros((rows, 1), jnp.float32)
    hi = jnp.full((rows, 1), 3.1e6, jnp.float32)
    z = jnp.zeros((rows, 1), jnp.float32)
    for _ in range(4):
        w = (hi - lo) * (1.0 / nb)
        edges = [lo + w * float(b + 1) for b in range(nb)]

        def hist_body(c, counts):
            chunk = d2_ref[:, pl.ds(c * cw, cw)]
            return tuple(
                counts[b] + jnp.sum((chunk <= edges[b]).astype(jnp.float32),
                                    axis=1, keepdims=True)
                for b in range(nb))
        counts = lax.fori_loop(0, nch, hist_body, (z,) * nb)

        new_lo, new_cnt = lo, cnt
        for b in range(nb):
            take = counts[b] < 19.5
            new_lo = jnp.where(take, edges[b], new_lo)
            new_cnt = jnp.where(take, counts[b], new_cnt)
        new_hi = hi
        for b in reversed(range(nb)):
            qual = counts[b] > 19.5
            new_hi = jnp.where(qual, edges[b], new_hi)
        lo, cnt, hi = new_lo, new_cnt, new_hi

    def ext_cond(carry):
        return jnp.any(carry[1] < 19.5)

    def ext_body(carry):
        r_cur, c_cur = carry

        def body(ch, acc):
            chunk = d2_ref[:, pl.ds(ch * cw, cw)]
            masked = jnp.where(chunk > r_cur, chunk, jnp.inf)
            return jnp.minimum(acc, jnp.min(masked, axis=1, keepdims=True))
        nxt = lax.fori_loop(0, nch, body,
                            jnp.full((rows, 1), jnp.inf, jnp.float32))
        need = c_cur < 19.5
        return (jnp.where(need, nxt, r_cur),
                jnp.where(need, c_cur + 1.0, c_cur))

    r, _ = lax.while_loop(ext_cond, ext_body, (lo, cnt))

    p1c = p1c_ref[...]
    p2c = p2c_ref[...]

    def cnt_body(c, carry):
        c1, c2 = carry
        sl = pl.ds(c * cw, cw)
        chunk = d2_ref[:, sl]
        nb = chunk <= r
        m1 = jnp.logical_and(nb, p1r_ref[:, sl] == p1c)
        m2 = jnp.logical_and(nb, p2r_ref[:, sl] == p2c)
        c1 = c1 + jnp.sum(m1.astype(jnp.float32), axis=1, keepdims=True)
        c2 = c2 + jnp.sum(m2.astype(jnp.float32), axis=1, keepdims=True)
        return c1, c2

    z = jnp.zeros((rows, 1), jnp.float32)
    c1, c2 = lax.fori_loop(0, nch, cnt_body, (z, z))
    ok = jnp.logical_and(c1 > 16.5, c2 > 16.5)
    mask_ref[...] = ok.astype(jnp.float32)


def _accum_body(feat_ref, w_ref, mask_ref, p2_ref, fh_ref, q_ref, g_ref):
    f = feat_ref[...]
    fh = jnp.maximum(
        lax.dot_general(f, w_ref[...], (((1,), (0,)), ((), ())),
                        precision=_PREC), 0.0)
    fh_ref[...] = fh
    fhm = fh * mask_ref[...]
    p2 = p2_ref[...]
    n = fh.shape[0]
    ncls = q_ref.shape[1]
    cls_iota = lax.broadcasted_iota(jnp.int32, (n, ncls), 1)
    yh = (cls_iota == p2).astype(jnp.float32)
    qc = lax.dot_general(fhm, yh, (((0,), (0,)), ((), ())), precision=_PREC)
    gc = lax.dot_general(fhm, fhm, (((0,), (0,)), ((), ())), precision=_PREC)

    @pl.when(pl.program_id(0) == 0)
    def _init():
        q_ref[...] = qc
        g_ref[...] = gc

    @pl.when(pl.program_id(0) != 0)
    def _acc():
        q_ref[...] = q_ref[...] + qc
        g_ref[...] = g_ref[...] + gc


def _out_body(fh_ref, w_ref, o_ref):
    o_ref[...] = lax.dot_general(fh_ref[...], w_ref[...],
                                 (((1,), (0,)), ((), ())), precision=_PREC)


def kernel(coords, feat, kernel, w_rand, proto):
    n, fd = feat.shape
    ncls = kernel.shape[1]
    h = w_rand.shape[1]
    na = proto.shape[0]

    classifier = kernel / (jnp.linalg.norm(kernel, axis=0, keepdims=True) + 1e-12)
    proto_n = proto / (jnp.linalg.norm(proto, axis=2, keepdims=True) + 1e-12)
    protof = proto_n.reshape(na * ncls, fd)

    p1, p2 = pl.pallas_call(
        _prologue_body,
        out_shape=(jax.ShapeDtypeStruct((n, 1), jnp.int32),
                   jax.ShapeDtypeStruct((n, 1), jnp.int32)),
    )(coords, feat, classifier, protof)

    rt = 256 if n % 256 == 0 else n
    cw = 512 if n % 512 == 0 else n
    coords_t = coords.T
    p1r = p1.reshape(1, n)
    p2r = p2.reshape(1, n)
    mask = pl.pallas_call(
        functools.partial(_knn_body, n_cols=n, cw=cw),
        grid=(n // rt,),
        in_specs=[
            pl.BlockSpec((rt, 3), lambda i: (i, 0)),
            pl.BlockSpec((3, n), lambda i: (0, 0)),
            pl.BlockSpec((1, n), lambda i: (0, 0)),
            pl.BlockSpec((1, n), lambda i: (0, 0)),
            pl.BlockSpec((rt, 1), lambda i: (i, 0)),
            pl.BlockSpec((rt, 1), lambda i: (i, 0)),
        ],
        out_specs=pl.BlockSpec((rt, 1), lambda i: (i, 0)),
        out_shape=jax.ShapeDtypeStruct((n, 1), jnp.float32),
        scratch_shapes=[pltpu.VMEM((rt, n), jnp.float32)],
    )(coords, coords_t, p1r, p2r, p1, p2)

    bt = 1024 if n % 1024 == 0 else n
    feat_h, q_mat, g_mat = pl.pallas_call(
        _accum_body,
        grid=(n // bt,),
        in_specs=[
            pl.BlockSpec((bt, fd), lambda i: (i, 0)),
            pl.BlockSpec((fd, h), lambda i: (0, 0)),
            pl.BlockSpec((bt, 1), lambda i: (i, 0)),
            pl.BlockSpec((bt, 1), lambda i: (i, 0)),
        ],
        out_specs=(pl.BlockSpec((bt, h), lambda i: (i, 0)),
                   pl.BlockSpec((h, ncls), lambda i: (0, 0)),
                   pl.BlockSpec((h, h), lambda i: (0, 0))),
        out_shape=(jax.ShapeDtypeStruct((n, h), jnp.float32),
                   jax.ShapeDtypeStruct((h, ncls), jnp.float32),
                   jax.ShapeDtypeStruct((h, h), jnp.float32)),
        compiler_params=pltpu.CompilerParams(
            dimension_semantics=("arbitrary",)),
    )(feat, w_rand, mask, p2)

    wo_t = jax.scipy.linalg.solve(
        g_mat + 100.0 * jnp.eye(h, dtype=jnp.float32), q_mat, assume_a='pos')

    pred_domain = pl.pallas_call(
        _out_body,
        grid=(n // bt,),
        in_specs=[pl.BlockSpec((bt, h), lambda i: (i, 0)),
                  pl.BlockSpec((h, ncls), lambda i: (0, 0))],
        out_specs=pl.BlockSpec((bt, ncls), lambda i: (i, 0)),
        out_shape=jax.ShapeDtypeStruct((n, ncls), jnp.float32),
    )(feat_h, wo_t)
    return pred_domain


# speedup vs baseline: 1.8009x; 1.8009x over previous
import functools

import jax
import jax.numpy as jnp
from jax import lax
from jax.experimental import pallas as pl
from jax.experimental.pallas import tpu as pltpu

_PREC = lax.Precision.HIGHEST
_KNN = 20
_E1 = float(1000.0 / 3.0)
_E2 = float(2000.0 / 3.0)


def _argmax_minor(x):
    n, c = x.shape
    mx = jnp.max(x, axis=1, keepdims=True)
    idx = lax.broadcasted_iota(jnp.int32, (n, c), 1).astype(jnp.float32)
    cand = jnp.where(x == mx, idx, float(c))
    return jnp.min(cand, axis=1, keepdims=True).astype(jnp.int32)


def _prologue_body(coords_ref, feat_ref, cls_ref, protof_ref, p1_ref, p2_ref):
    f = feat_ref[...]
    logits1 = lax.dot_general(f, cls_ref[...], (((1,), (0,)), ((), ())),
                              precision=_PREC)
    p1_ref[...] = _argmax_minor(logits1)

    c = coords_ref[...]
    d = jnp.sqrt(c[:, 0:1] * c[:, 0:1] + c[:, 1:2] * c[:, 1:2])
    d = jnp.clip(d, 0.001, 999.999)
    lab = (d >= _E1).astype(jnp.int32) + (d >= _E2).astype(jnp.int32)

    la = lax.dot_general(f, protof_ref[...], (((1,), (1,)), ((), ())),
                         precision=_PREC)
    nc = la.shape[1] // 3
    pp = jnp.where(lab == 0, la[:, 0:nc],
                   jnp.where(lab == 1, la[:, nc:2 * nc], la[:, 2 * nc:3 * nc]))
    p2_ref[...] = _argmax_minor(pp)


def _knn_body(coords_r_ref, coords_t_ref, p1r_ref, p2r_ref, p1c_ref, p2c_ref,
              mask_ref, d2_ref, *, n_cols, cw):
    cr = coords_r_ref[...]
    ct = coords_t_ref[...]
    sq_r = jnp.sum(cr * cr, axis=1, keepdims=True)
    sq_c = jnp.sum(ct * ct, axis=0, keepdims=True)
    dot = lax.dot_general(cr, ct, (((1,), (0,)), ((), ())), precision=_PREC)
    d2_ref[...] = sq_r + sq_c - 2.0 * dot

    rows = cr.shape[0]
    nch = n_cols // cw

    def min_pass(_, r_cur):
        def body(c, acc):
            chunk = d2_ref[:, pl.ds(c * cw, cw)]
            masked = jnp.where(chunk > r_cur, chunk, jnp.inf)
            return jnp.minimum(acc, jnp.min(masked, axis=1, keepdims=True))
        return lax.fori_loop(0, nch, body,
                             jnp.full((rows, 1), jnp.inf, jnp.float32))

    r0 = jnp.full((rows, 1), -jnp.inf, jnp.float32)
    r = lax.fori_loop(0, _KNN, min_pass, r0)

    p1c = p1c_ref[...]
    p2c = p2c_ref[...]

    def cnt_body(c, carry):
        c1, c2 = carry
        sl = pl.ds(c * cw, cw)
        chunk = d2_ref[:, sl]
        nb = chunk <= r
        m1 = jnp.logical_and(nb, p1r_ref[:, sl] == p1c)
        m2 = jnp.logical_and(nb, p2r_ref[:, sl] == p2c)
        c1 = c1 + jnp.sum(m1.astype(jnp.float32), axis=1, keepdims=True)
        c2 = c2 + jnp.sum(m2.astype(jnp.float32), axis=1, keepdims=True)
        return c1, c2

    z = jnp.zeros((rows, 1), jnp.float32)
    c1, c2 = lax.fori_loop(0, nch, cnt_body, (z, z))
    ok = jnp.logical_and(c1 > 16.5, c2 > 16.5)
    mask_ref[...] = ok.astype(jnp.float32)


def _accum_body(feat_ref, w_ref, mask_ref, p2_ref, fh_ref, q_ref, g_ref):
    f = feat_ref[...]
    fh = jnp.maximum(
        lax.dot_general(f, w_ref[...], (((1,), (0,)), ((), ())),
                        precision=_PREC), 0.0)
    fh_ref[...] = fh
    fhm = fh * mask_ref[...]
    p2 = p2_ref[...]
    n = fh.shape[0]
    ncls = q_ref.shape[1]
    cls_iota = lax.broadcasted_iota(jnp.int32, (n, ncls), 1)
    yh = (cls_iota == p2).astype(jnp.float32)
    qc = lax.dot_general(fhm, yh, (((0,), (0,)), ((), ())), precision=_PREC)
    gc = lax.dot_general(fhm, fhm, (((0,), (0,)), ((), ())), precision=_PREC)

    @pl.when(pl.program_id(0) == 0)
    def _init():
        q_ref[...] = qc
        g_ref[...] = gc

    @pl.when(pl.program_id(0) != 0)
    def _acc():
        q_ref[...] = q_ref[...] + qc
        g_ref[...] = g_ref[...] + gc


def _out_body(fh_ref, w_ref, o_ref):
    o_ref[...] = lax.dot_general(fh_ref[...], w_ref[...],
                                 (((1,), (0,)), ((), ())), precision=_PREC)


def kernel(coords, feat, kernel, w_rand, proto):
    n, fd = feat.shape
    ncls = kernel.shape[1]
    h = w_rand.shape[1]
    na = proto.shape[0]

    classifier = kernel / (jnp.linalg.norm(kernel, axis=0, keepdims=True) + 1e-12)
    proto_n = proto / (jnp.linalg.norm(proto, axis=2, keepdims=True) + 1e-12)
    protof = proto_n.reshape(na * ncls, fd)

    p1, p2 = pl.pallas_call(
        _prologue_body,
        out_shape=(jax.ShapeDtypeStruct((n, 1), jnp.int32),
                   jax.ShapeDtypeStruct((n, 1), jnp.int32)),
    )(coords, feat, classifier, protof)

    rt = 512 if n % 512 == 0 else n
    cw = 512 if n % 512 == 0 else n
    coords_t = coords.T
    p1r = p1.reshape(1, n)
    p2r = p2.reshape(1, n)
    mask = pl.pallas_call(
        functools.partial(_knn_body, n_cols=n, cw=cw),
        grid=(n // rt,),
        in_specs=[
            pl.BlockSpec((rt, 3), lambda i: (i, 0)),
            pl.BlockSpec((3, n), lambda i: (0, 0)),
            pl.BlockSpec((1, n), lambda i: (0, 0)),
            pl.BlockSpec((1, n), lambda i: (0, 0)),
            pl.BlockSpec((rt, 1), lambda i: (i, 0)),
            pl.BlockSpec((rt, 1), lambda i: (i, 0)),
        ],
        out_specs=pl.BlockSpec((rt, 1), lambda i: (i, 0)),
        out_shape=jax.ShapeDtypeStruct((n, 1), jnp.float32),
        scratch_shapes=[pltpu.VMEM((rt, n), jnp.float32)],
    )(coords, coords_t, p1r, p2r, p1, p2)

    bt = 1024 if n % 1024 == 0 else n
    feat_h, q_mat, g_mat = pl.pallas_call(
        _accum_body,
        grid=(n // bt,),
        in_specs=[
            pl.BlockSpec((bt, fd), lambda i: (i, 0)),
            pl.BlockSpec((fd, h), lambda i: (0, 0)),
            pl.BlockSpec((bt, 1), lambda i: (i, 0)),
            pl.BlockSpec((bt, 1), lambda i: (i, 0)),
        ],
        out_specs=(pl.BlockSpec((bt, h), lambda i: (i, 0)),
                   pl.BlockSpec((h, ncls), lambda i: (0, 0)),
                   pl.BlockSpec((h, h), lambda i: (0, 0))),
        out_shape=(jax.ShapeDtypeStruct((n, h), jnp.float32),
                   jax.ShapeDtypeStruct((h, ncls), jnp.float32),
                   jax.ShapeDtypeStruct((h, h), jnp.float32)),
        compiler_params=pltpu.CompilerParams(
            dimension_semantics=("arbitrary",)),
    )(feat, w_rand, mask, p2)

    wo_t = jax.scipy.linalg.solve(
        g_mat + 100.0 * jnp.eye(h, dtype=jnp.float32), q_mat, assume_a='pos')

    pred_domain = pl.pallas_call(
        _out_body,
        grid=(n // bt,),
        in_specs=[pl.BlockSpec((bt, h), lambda i: (i, 0)),
                  pl.BlockSpec((h, ncls), lambda i: (0, 0))],
        out_specs=pl.BlockSpec((bt, ncls), lambda i: (i, 0)),
        out_shape=jax.ShapeDtypeStruct((n, ncls), jnp.float32),
    )(feat_h, wo_t)
    return pred_domain


# speedup vs baseline: 1.9351x; 1.0746x over previous
import functools

import jax
import jax.numpy as jnp
from jax import lax
from jax.experimental import pallas as pl
from jax.experimental.pallas import tpu as pltpu

_PREC = lax.Precision.HIGHEST
_KNN = 20
_E1 = float(1000.0 / 3.0)
_E2 = float(2000.0 / 3.0)


def _argmax_minor(x):
    n, c = x.shape
    mx = jnp.max(x, axis=1, keepdims=True)
    idx = lax.broadcasted_iota(jnp.int32, (n, c), 1).astype(jnp.float32)
    cand = jnp.where(x == mx, idx, float(c))
    return jnp.min(cand, axis=1, keepdims=True).astype(jnp.int32)


def _prologue_body(coords_ref, feat_ref, cls_ref, protof_ref, p1_ref, p2_ref):
    f = feat_ref[...]
    logits1 = lax.dot_general(f, cls_ref[...], (((1,), (0,)), ((), ())),
                              precision=_PREC)
    p1_ref[...] = _argmax_minor(logits1)

    c = coords_ref[...]
    d = jnp.sqrt(c[:, 0:1] * c[:, 0:1] + c[:, 1:2] * c[:, 1:2])
    d = jnp.clip(d, 0.001, 999.999)
    lab = (d >= _E1).astype(jnp.int32) + (d >= _E2).astype(jnp.int32)

    la = lax.dot_general(f, protof_ref[...], (((1,), (1,)), ((), ())),
                         precision=_PREC)
    nc = la.shape[1] // 3
    pp = jnp.where(lab == 0, la[:, 0:nc],
                   jnp.where(lab == 1, la[:, nc:2 * nc], la[:, 2 * nc:3 * nc]))
    p2_ref[...] = _argmax_minor(pp)


def _knn_body(coords_r_ref, coords_t_ref, p1r_ref, p2r_ref, p1c_ref, p2c_ref,
              mask_ref, d2_ref, *, n_cols, cw):
    cr = coords_r_ref[...]
    ct = coords_t_ref[...]
    sq_r = jnp.sum(cr * cr, axis=1, keepdims=True)
    sq_c = jnp.sum(ct * ct, axis=0, keepdims=True)
    dot = lax.dot_general(cr, ct, (((1,), (0,)), ((), ())), precision=_PREC)
    d2_ref[...] = sq_r + sq_c - 2.0 * dot

    rows = cr.shape[0]
    nch = n_cols // cw

    def min_pass(_, r_cur):
        def body(c, acc):
            chunk = d2_ref[:, pl.ds(c * cw, cw)]
            masked = jnp.where(chunk > r_cur, chunk, jnp.inf)
            return jnp.minimum(acc, jnp.min(masked, axis=1, keepdims=True))
        return lax.fori_loop(0, nch, body,
                             jnp.full((rows, 1), jnp.inf, jnp.float32))

    r0 = jnp.full((rows, 1), -jnp.inf, jnp.float32)
    r = lax.fori_loop(0, _KNN, min_pass, r0)

    p1c = p1c_ref[...]
    p2c = p2c_ref[...]

    def cnt_body(c, carry):
        c1, c2 = carry
        sl = pl.ds(c * cw, cw)
        chunk = d2_ref[:, sl]
        nb = chunk <= r
        m1 = jnp.logical_and(nb, p1r_ref[:, sl] == p1c)
        m2 = jnp.logical_and(nb, p2r_ref[:, sl] == p2c)
        c1 = c1 + jnp.sum(m1.astype(jnp.float32), axis=1, keepdims=True)
        c2 = c2 + jnp.sum(m2.astype(jnp.float32), axis=1, keepdims=True)
        return c1, c2

    z = jnp.zeros((rows, 1), jnp.float32)
    c1, c2 = lax.fori_loop(0, nch, cnt_body, (z, z))
    ok = jnp.logical_and(c1 > 16.5, c2 > 16.5)
    mask_ref[...] = ok.astype(jnp.float32)


def _accum_body(feat_ref, w_ref, mask_ref, p2_ref, fh_ref, q_ref, g_ref):
    f = feat_ref[...]
    fh = jnp.maximum(
        lax.dot_general(f, w_ref[...], (((1,), (0,)), ((), ())),
                        precision=_PREC), 0.0)
    fh_ref[...] = fh
    fhm = fh * mask_ref[...]
    p2 = p2_ref[...]
    n = fh.shape[0]
    ncls = q_ref.shape[1]
    cls_iota = lax.broadcasted_iota(jnp.int32, (n, ncls), 1)
    yh = (cls_iota == p2).astype(jnp.float32)
    qc = lax.dot_general(fhm, yh, (((0,), (0,)), ((), ())), precision=_PREC)
    gc = lax.dot_general(fhm, fhm, (((0,), (0,)), ((), ())), precision=_PREC)

    @pl.when(pl.program_id(0) == 0)
    def _init():
        q_ref[...] = qc
        g_ref[...] = gc

    @pl.when(pl.program_id(0) != 0)
    def _acc():
        q_ref[...] = q_ref[...] + qc
        g_ref[...] = g_ref[...] + gc


def _out_body(fh_ref, w_ref, o_ref):
    o_ref[...] = lax.dot_general(fh_ref[...], w_ref[...],
                                 (((1,), (0,)), ((), ())), precision=_PREC)


def kernel(coords, feat, kernel, w_rand, proto):
    n, fd = feat.shape
    ncls = kernel.shape[1]
    h = w_rand.shape[1]
    na = proto.shape[0]

    classifier = kernel / (jnp.linalg.norm(kernel, axis=0, keepdims=True) + 1e-12)
    proto_n = proto / (jnp.linalg.norm(proto, axis=2, keepdims=True) + 1e-12)
    protof = proto_n.reshape(na * ncls, fd)

    p1, p2 = pl.pallas_call(
        _prologue_body,
        out_shape=(jax.ShapeDtypeStruct((n, 1), jnp.int32),
                   jax.ShapeDtypeStruct((n, 1), jnp.int32)),
    )(coords, feat, classifier, protof)

    rt = 1024 if n % 1024 == 0 else n
    cw = 512 if n % 512 == 0 else n
    coords_t = coords.T
    p1r = p1.reshape(1, n)
    p2r = p2.reshape(1, n)
    mask = pl.pallas_call(
        functools.partial(_knn_body, n_cols=n, cw=cw),
        grid=(n // rt,),
        in_specs=[
            pl.BlockSpec((rt, 3), lambda i: (i, 0)),
            pl.BlockSpec((3, n), lambda i: (0, 0)),
            pl.BlockSpec((1, n), lambda i: (0, 0)),
            pl.BlockSpec((1, n), lambda i: (0, 0)),
            pl.BlockSpec((rt, 1), lambda i: (i, 0)),
            pl.BlockSpec((rt, 1), lambda i: (i, 0)),
        ],
        out_specs=pl.BlockSpec((rt, 1), lambda i: (i, 0)),
        out_shape=jax.ShapeDtypeStruct((n, 1), jnp.float32),
        scratch_shapes=[pltpu.VMEM((rt, n), jnp.float32)],
    )(coords, coords_t, p1r, p2r, p1, p2)

    bt = 1024 if n % 1024 == 0 else n
    feat_h, q_mat, g_mat = pl.pallas_call(
        _accum_body,
        grid=(n // bt,),
        in_specs=[
            pl.BlockSpec((bt, fd), lambda i: (i, 0)),
            pl.BlockSpec((fd, h), lambda i: (0, 0)),
            pl.BlockSpec((bt, 1), lambda i: (i, 0)),
            pl.BlockSpec((bt, 1), lambda i: (i, 0)),
        ],
        out_specs=(pl.BlockSpec((bt, h), lambda i: (i, 0)),
                   pl.BlockSpec((h, ncls), lambda i: (0, 0)),
                   pl.BlockSpec((h, h), lambda i: (0, 0))),
        out_shape=(jax.ShapeDtypeStruct((n, h), jnp.float32),
                   jax.ShapeDtypeStruct((h, ncls), jnp.float32),
                   jax.ShapeDtypeStruct((h, h), jnp.float32)),
        compiler_params=pltpu.CompilerParams(
            dimension_semantics=("arbitrary",)),
    )(feat, w_rand, mask, p2)

    wo_t = jax.scipy.linalg.solve(
        g_mat + 100.0 * jnp.eye(h, dtype=jnp.float32), q_mat, assume_a='pos')

    pred_domain = pl.pallas_call(
        _out_body,
        grid=(n // bt,),
        in_specs=[pl.BlockSpec((bt, h), lambda i: (i, 0)),
                  pl.BlockSpec((h, ncls), lambda i: (0, 0))],
        out_specs=pl.BlockSpec((bt, ncls), lambda i: (i, 0)),
        out_shape=jax.ShapeDtypeStruct((n, ncls), jnp.float32),
    )(feat_h, wo_t)
    return pred_domain
